# baseline (speedup 1.0000x reference)
import functools
import math

import jax
import jax.numpy as jnp
from jax import lax
from jax.experimental import pallas as pl
from jax.experimental.pallas import tpu as pltpu
from jax.experimental.pallas import tpu_sc as plsc

_L = 16
_BB = 2048


def _make_sc_prep(B_pad, S, n_workers, nc):
    rows_per_w = B_pad // n_workers
    C = 8
    for c in range(8, 225, 8):
        if rows_per_w % c == 0 and (c * S) % 8 == 0:
            C = c
    n_chunks = rows_per_w // C

    mesh = plsc.VectorSubcoreMesh(core_axis_name="c", subcore_axis_name="s")

    @functools.partial(
        pl.kernel,
        mesh=mesh,
        out_type=(
            jax.ShapeDtypeStruct((B_pad * S,), jnp.int32),
            jax.ShapeDtypeStruct((B_pad * 2,), jnp.float32),
        ),
        scratch_types=[
            pltpu.VMEM((C, _L), jnp.int32),
            pltpu.VMEM((C * S + _L,), jnp.int32),
            pltpu.VMEM((C * 2 + _L,), jnp.float32),
        ],
        compiler_params=pltpu.CompilerParams(needs_layout_passes=False),
    )
    def sc_prep(nidx_hbm, g_hbm, r_hbm, nidx_v, g_v, r_v):
        wid = lax.axis_index("s") * nc + lax.axis_index("c")
        iota = lax.iota(jnp.int32, _L)
        elig = iota < S
        zero_v = jnp.zeros((_L,), jnp.int32)
        s_f = jnp.float32(S)

        def chunk_body(ci, _):
            base = wid * rows_per_w + ci * C
            pltpu.sync_copy(nidx_hbm.at[pl.ds(base, C)], nidx_v)

            def row_body(r, _):
                x = nidx_v[r]
                _, last = plsc.scan_count(x, mask=elig)
                cnt = plsc.all_reduce_population_count(last)
                g_v[pl.ds(r * S, _L)] = zero_v
                plsc.store_compressed(g_v.at[pl.ds(r * S, _L)], x, mask=last)
                a = 1.0 / jnp.maximum(cnt, 1).astype(jnp.float32)
                b = (cnt.astype(jnp.float32) - s_f) * a
                r_v[pl.ds(r * 2, _L)] = jnp.where(iota == 0, a, b)
                return 0

            lax.fori_loop(0, C, row_body, 0, unroll=False)

            pltpu.sync_copy(g_v.at[pl.ds(0, C * S)],
                            g_hbm.at[pl.ds(base * S, C * S)])
            pltpu.sync_copy(r_v.at[pl.ds(0, C * 2)],
                            r_hbm.at[pl.ds(base * 2, C * 2)])
            return 0

        lax.fori_loop(0, n_chunks, chunk_body, 0, unroll=False)

    return sc_prep


def _make_tc_gather(B, B_pad, N, D, S):
    grid = (B_pad // _BB,)

    def tc_body(g_smem, r_smem, emb_hbm, out_vmem, emb_vmem, sem):
        @pl.when(pl.program_id(0) == 0)
        def _stage():
            pltpu.make_async_copy(emb_hbm, emb_vmem, sem).start()
            pltpu.make_async_copy(emb_hbm, emb_vmem, sem).wait()

        row0 = emb_vmem[0:1, :]

        def row_body(r, _):
            gb = r * S
            acc = emb_vmem[pl.ds(g_smem[gb], 1), :]
            for s in range(1, S):
                acc = acc + emb_vmem[pl.ds(g_smem[gb + s], 1), :]
            out_vmem[pl.ds(r, 1), :] = (
                acc * r_smem[r * 2] + row0 * r_smem[r * 2 + 1])
            return 0

        lax.fori_loop(0, _BB, row_body, 0, unroll=128)

    return pl.pallas_call(
        tc_body,
        grid=grid,
        in_specs=[
            pl.BlockSpec((_BB * S,), lambda i: (i,),
                         memory_space=pltpu.SMEM),
            pl.BlockSpec((_BB * 2,), lambda i: (i,),
                         memory_space=pltpu.SMEM),
            pl.BlockSpec(memory_space=pl.ANY),
        ],
        out_specs=pl.BlockSpec((_BB, D), lambda i: (i, 0)),
        out_shape=jax.ShapeDtypeStruct((B, D), jnp.float32),
        scratch_shapes=[
            pltpu.VMEM((N, D), jnp.float32),
            pltpu.SemaphoreType.DMA,
        ],
        compiler_params=pltpu.CompilerParams(
            dimension_semantics=("arbitrary",),
        ),
    )


def kernel(nodes, neigh_idx, emb):
    del nodes
    B, S = neigh_idx.shape
    N, D = emb.shape

    info = plsc.get_sparse_core_info()
    nw = info.num_cores * info.num_subcores

    step = math.lcm(nw, _BB)
    B_pad = (B + step - 1) // step * step

    idx = neigh_idx.astype(jnp.int32)
    idx = jnp.pad(idx, ((0, B_pad - B), (0, _L - S)))

    g_flat, r_flat = _make_sc_prep(B_pad, S, nw, info.num_cores)(idx)
    return _make_tc_gather(B, B_pad, N, D, S)(g_flat, r_flat, emb)


# speedup vs baseline: 1.0134x; 1.0134x over previous
import functools
import math

import jax
import jax.numpy as jnp
from jax import lax
from jax.experimental import pallas as pl
from jax.experimental.pallas import tpu as pltpu
from jax.experimental.pallas import tpu_sc as plsc

_L = 16
_BB = 1024


def _make_sc_prep(B_pad, S, n_workers, nc):
    rows_per_w = B_pad // n_workers
    C = 8
    for c in range(8, 225, 8):
        if rows_per_w % c == 0 and (c * S) % 8 == 0:
            C = c
    n_chunks = rows_per_w // C

    mesh = plsc.VectorSubcoreMesh(core_axis_name="c", subcore_axis_name="s")

    @functools.partial(
        pl.kernel,
        mesh=mesh,
        out_type=(
            jax.ShapeDtypeStruct((B_pad * S,), jnp.int32),
            jax.ShapeDtypeStruct((B_pad * 2,), jnp.float32),
        ),
        scratch_types=[
            pltpu.VMEM((C, _L), jnp.int32),
            pltpu.VMEM((C * S + _L,), jnp.int32),
            pltpu.VMEM((C * 2 + _L,), jnp.float32),
        ],
        compiler_params=pltpu.CompilerParams(needs_layout_passes=False),
    )
    def sc_prep(nidx_hbm, g_hbm, r_hbm, nidx_v, g_v, r_v):
        wid = lax.axis_index("s") * nc + lax.axis_index("c")
        iota = lax.iota(jnp.int32, _L)
        elig = iota < S
        zero_v = jnp.zeros((_L,), jnp.int32)
        s_f = jnp.float32(S)

        def chunk_body(ci, _):
            base = wid * rows_per_w + ci * C
            pltpu.sync_copy(nidx_hbm.at[pl.ds(base, C)], nidx_v)

            def row_body(r, _):
                x = nidx_v[r]
                _, last = plsc.scan_count(x, mask=elig)
                cnt = plsc.all_reduce_population_count(last)
                g_v[pl.ds(r * S, _L)] = zero_v
                plsc.store_compressed(g_v.at[pl.ds(r * S, _L)], x, mask=last)
                a = 1.0 / jnp.maximum(cnt, 1).astype(jnp.float32)
                b = (cnt.astype(jnp.float32) - s_f) * a
                r_v[pl.ds(r * 2, _L)] = jnp.where(iota == 0, a, b)
                return 0

            lax.fori_loop(0, C, row_body, 0, unroll=False)

            pltpu.sync_copy(g_v.at[pl.ds(0, C * S)],
                            g_hbm.at[pl.ds(base * S, C * S)])
            pltpu.sync_copy(r_v.at[pl.ds(0, C * 2)],
                            r_hbm.at[pl.ds(base * 2, C * 2)])
            return 0

        lax.fori_loop(0, n_chunks, chunk_body, 0, unroll=False)

    return sc_prep


def _make_tc_gather(B, B_pad, N, D, S):
    grid = (B_pad // _BB,)

    def tc_body(g_smem, r_smem, emb_hbm, out_vmem, emb_vmem, sem):
        @pl.when(pl.program_id(0) == 0)
        def _stage():
            pltpu.make_async_copy(emb_hbm, emb_vmem, sem).start()
            pltpu.make_async_copy(emb_hbm, emb_vmem, sem).wait()

        row0 = emb_vmem[0:1, :]

        def row_body(r, _):
            gb = r * S
            acc = emb_vmem[pl.ds(g_smem[gb], 1), :]
            for s in range(1, S):
                acc = acc + emb_vmem[pl.ds(g_smem[gb + s], 1), :]
            out_vmem[pl.ds(r, 1), :] = (
                acc * r_smem[r * 2] + row0 * r_smem[r * 2 + 1])
            return 0

        lax.fori_loop(0, _BB, row_body, 0, unroll=64)

    return pl.pallas_call(
        tc_body,
        grid=grid,
        in_specs=[
            pl.BlockSpec((_BB * S,), lambda i: (i,),
                         memory_space=pltpu.SMEM),
            pl.BlockSpec((_BB * 2,), lambda i: (i,),
                         memory_space=pltpu.SMEM),
            pl.BlockSpec(memory_space=pl.ANY),
        ],
        out_specs=pl.BlockSpec((_BB, D), lambda i: (i, 0)),
        out_shape=jax.ShapeDtypeStruct((B, D), jnp.float32),
        scratch_shapes=[
            pltpu.VMEM((N, D), jnp.float32),
            pltpu.SemaphoreType.DMA,
        ],
        compiler_params=pltpu.CompilerParams(
            dimension_semantics=("arbitrary",),
        ),
    )


def kernel(nodes, neigh_idx, emb):
    del nodes
    B, S = neigh_idx.shape
    N, D = emb.shape

    info = plsc.get_sparse_core_info()
    nw = info.num_cores * info.num_subcores

    step = math.lcm(nw, _BB)
    B_pad = (B + step - 1) // step * step

    idx = neigh_idx.astype(jnp.int32)
    idx = jnp.pad(idx, ((0, B_pad - B), (0, _L - S)))

    g_flat, r_flat = _make_sc_prep(B_pad, S, nw, info.num_cores)(idx)
    return _make_tc_gather(B, B_pad, N, D, S)(g_flat, r_flat, emb)
